# Initial kernel scaffold; baseline (speedup 1.0000x reference)
import functools

import jax
import jax.numpy as jnp
from jax import lax
from jax.experimental import pallas as pl
from jax.experimental.pallas import tpu as pltpu
from jax.experimental.pallas import tpu_sc as plsc

WORD_DIM = 64
SMALL_DIM = 16
OUT_DIM = WORD_DIM + 3 * SMALL_DIM

NUM_WORKERS = 32
GRP = 128
GRPS_PER_CHUNK = 5
CHUNK = GRP * GRPS_PER_CHUNK


def _sc_body(word_idx, bio_idx, f0_idx, f1_idx, w_word, w_bio, w_feat, out,
             idx_w, idx_b, idx_0, idx_1, rows_w, rows_b, rows_0, rows_1, sem):
    n_total = word_idx.shape[0]
    per_worker = n_total // NUM_WORKERS
    n_chunks = per_worker // CHUNK

    wid = lax.axis_index("s") * 2 + lax.axis_index("c")
    worker_base = wid * per_worker

    def chunk_body(c, carry):
        base = worker_base + c * CHUNK
        pltpu.sync_copy(word_idx.at[pl.ds(base, CHUNK)], idx_w)
        pltpu.sync_copy(bio_idx.at[pl.ds(base, CHUNK)], idx_b)
        pltpu.sync_copy(f0_idx.at[pl.ds(base, CHUNK)], idx_0)
        pltpu.sync_copy(f1_idx.at[pl.ds(base, CHUNK)], idx_1)

        copies = []
        for j in range(GRPS_PER_CHUNK):
            sl = pl.ds(j * GRP, GRP)
            copies.append(pltpu.async_copy(
                w_word.at[idx_w.at[sl]], rows_w.at[sl], sem))
            copies.append(pltpu.async_copy(
                w_bio.at[idx_b.at[sl]], rows_b.at[sl], sem))
            copies.append(pltpu.async_copy(
                w_feat.at[idx_0.at[sl]], rows_0.at[sl], sem))
            copies.append(pltpu.async_copy(
                w_feat.at[idx_1.at[sl]], rows_1.at[sl], sem))
        for cp in copies:
            cp.wait()

        out_rows = pl.ds(base, CHUNK)
        pltpu.sync_copy(rows_w, out.at[out_rows, pl.ds(0, WORD_DIM)])
        pltpu.sync_copy(rows_b, out.at[out_rows, pl.ds(WORD_DIM, SMALL_DIM)])
        pltpu.sync_copy(rows_0, out.at[out_rows, pl.ds(WORD_DIM + SMALL_DIM, SMALL_DIM)])
        pltpu.sync_copy(rows_1, out.at[out_rows, pl.ds(WORD_DIM + 2 * SMALL_DIM, SMALL_DIM)])
        return carry

    lax.fori_loop(0, n_chunks, chunk_body, 0)


def kernel(word_index, bio_index, feat_index_0, feat_index_1, W_word, W_bio, W_feat):
    B, S = word_index.shape
    n = B * S
    wf = word_index.reshape(n).astype(jnp.int32)
    bf = bio_index.reshape(n).astype(jnp.int32)
    f0 = feat_index_0.reshape(n).astype(jnp.int32)
    f1 = feat_index_1.reshape(n).astype(jnp.int32)

    mesh = plsc.VectorSubcoreMesh(core_axis_name="c", subcore_axis_name="s")
    run = pl.kernel(
        _sc_body,
        out_type=jax.ShapeDtypeStruct((n, OUT_DIM), jnp.float32),
        mesh=mesh,
        scratch_types=[
            pltpu.VMEM((CHUNK,), jnp.int32),
            pltpu.VMEM((CHUNK,), jnp.int32),
            pltpu.VMEM((CHUNK,), jnp.int32),
            pltpu.VMEM((CHUNK,), jnp.int32),
            pltpu.VMEM((CHUNK, WORD_DIM), jnp.float32),
            pltpu.VMEM((CHUNK, SMALL_DIM), jnp.float32),
            pltpu.VMEM((CHUNK, SMALL_DIM), jnp.float32),
            pltpu.VMEM((CHUNK, SMALL_DIM), jnp.float32),
            pltpu.SemaphoreType.DMA,
        ],
    )
    out = run(wf, bf, f0, f1, W_word, W_bio, W_feat)
    return out.reshape(B, S, OUT_DIM)


# speedup vs baseline: 1.2136x; 1.2136x over previous
import jax
import jax.numpy as jnp
from jax import lax
from jax.experimental import pallas as pl
from jax.experimental.pallas import tpu as pltpu
from jax.experimental.pallas import tpu_sc as plsc

WORD_DIM = 64
SMALL_DIM = 16
OUT_DIM = WORD_DIM + 3 * SMALL_DIM

NUM_WORKERS = 32
GRP = 128
GRPS_PER_CHUNK = 5
CHUNK = GRP * GRPS_PER_CHUNK


def _sc_body(word_idx, bio_idx, f0_idx, f1_idx, w_word, w_bio, w_feat, out,
             *scratch):
    k = GRPS_PER_CHUNK
    idx_w = scratch[0:k]
    idx_b = scratch[k:2 * k]
    idx_0 = scratch[2 * k:3 * k]
    idx_1 = scratch[3 * k:4 * k]
    rows_w, rows_b, rows_0, rows_1, sem = scratch[4 * k:]

    n_total = word_idx.shape[0]
    per_worker = n_total // NUM_WORKERS
    n_chunks = per_worker // CHUNK

    sid = lax.axis_index("s")
    wid = sid * 2 + lax.axis_index("c")
    worker_base = wid * per_worker

    def chunk_body(c, carry):
        base = worker_base + c * CHUNK
        idx_loads = []
        for j in range(k):
            gsl = pl.ds(base + j * GRP, GRP)
            idx_loads.append(pltpu.async_copy(word_idx.at[gsl], idx_w[j], sem))
            idx_loads.append(pltpu.async_copy(bio_idx.at[gsl], idx_b[j], sem))
            idx_loads.append(pltpu.async_copy(f0_idx.at[gsl], idx_0[j], sem))
            idx_loads.append(pltpu.async_copy(f1_idx.at[gsl], idx_1[j], sem))
        for cp in idx_loads:
            cp.wait()

        gathers = []
        for j in range(k):
            sl = pl.ds(j * GRP, GRP)
            gathers.append(pltpu.async_copy(
                w_word.at[idx_w[j]], rows_w.at[sl], sem))
            gathers.append(pltpu.async_copy(
                w_bio.at[idx_b[j]], rows_b.at[sl], sem))
            gathers.append(pltpu.async_copy(
                w_feat.at[idx_0[j]], rows_0.at[sl], sem))
            gathers.append(pltpu.async_copy(
                w_feat.at[idx_1[j]], rows_1.at[sl], sem))
        for cp in gathers:
            cp.wait()

        out_rows = pl.ds(base, CHUNK)
        pltpu.sync_copy(rows_w, out.at[out_rows, pl.ds(0, WORD_DIM)])
        pltpu.sync_copy(rows_b, out.at[out_rows, pl.ds(WORD_DIM, SMALL_DIM)])
        pltpu.sync_copy(rows_0, out.at[out_rows, pl.ds(WORD_DIM + SMALL_DIM, SMALL_DIM)])
        pltpu.sync_copy(rows_1, out.at[out_rows, pl.ds(WORD_DIM + 2 * SMALL_DIM, SMALL_DIM)])
        return carry

    lax.fori_loop(0, n_chunks, chunk_body, 0)


def kernel(word_index, bio_index, feat_index_0, feat_index_1, W_word, W_bio, W_feat):
    B, S = word_index.shape
    n = B * S
    wf = word_index.reshape(n).astype(jnp.int32)
    bf = bio_index.reshape(n).astype(jnp.int32)
    f0 = feat_index_0.reshape(n).astype(jnp.int32)
    f1 = feat_index_1.reshape(n).astype(jnp.int32)

    mesh = plsc.VectorSubcoreMesh(
        core_axis_name="c", subcore_axis_name="s", num_cores=2, num_subcores=16)
    idx_scratch = [pltpu.VMEM((GRP,), jnp.int32)
                   for _ in range(4 * GRPS_PER_CHUNK)]
    run = pl.kernel(
        _sc_body,
        out_type=jax.ShapeDtypeStruct((n, OUT_DIM), jnp.float32),
        mesh=mesh,
        scratch_types=idx_scratch + [
            pltpu.VMEM((CHUNK, WORD_DIM), jnp.float32),
            pltpu.VMEM((CHUNK, SMALL_DIM), jnp.float32),
            pltpu.VMEM((CHUNK, SMALL_DIM), jnp.float32),
            pltpu.VMEM((CHUNK, SMALL_DIM), jnp.float32),
            pltpu.SemaphoreType.DMA,
        ],
        compiler_params=pltpu.CompilerParams(use_tc_tiling_on_sc=False),
    )
    out = run(wf, bf, f0, f1, W_word, W_bio, W_feat)
    return out.reshape(B, S, OUT_DIM)
